# baseline (speedup 1.0000x reference)
import functools

import jax
import jax.numpy as jnp
from jax import lax
from jax.experimental import pallas as pl
from jax.experimental.pallas import tpu as pltpu
from jax.experimental.pallas import tpu_sc as plsc

RISE_IDX = 0
VIS_IDX = 2

NC = 2
NS = 16
L = 16
NW = NC * NS

BLK_V = 8
BLK = BLK_V * L


def _sc_body(step, batch, seq, feat,
             raw_hbm, dist_hbm, pt_hbm, pa_hbm, fa_hbm, bi_hbm,
             time_out, act_out, mask_out,
             bi_v, pa_v, fa_v, pt_v,
             idx_dist, idx_rise, idx_dur, dist_vals, rise_vals, dur_vals,
             tbuf, abuf, mbuf, sem0, sem1, sem2):
    stripe = batch // NW
    wid = lax.axis_index("s") * NC + lax.axis_index("c")
    wlo = wid * stripe
    whi = wlo + stripe
    pad = bi_v.shape[0] - step

    cp_bi = pltpu.async_copy(bi_hbm, bi_v.at[pl.ds(0, step)], sem0)
    cp_pa = pltpu.async_copy(pa_hbm, pa_v.at[pl.ds(0, step)], sem0)
    cp_fa = pltpu.async_copy(fa_hbm, fa_v.at[pl.ds(0, step)], sem0)
    cp_pt = pltpu.async_copy(pt_hbm, pt_v.at[pl.ds(0, step)], sem0)

    zeros_i = jnp.zeros((L,), jnp.int32)
    zeros_f = jnp.zeros((L,), jnp.float32)
    ones_i = jnp.ones((L,), jnp.int32)
    iota = lax.iota(jnp.int32, L)

    for k in range(stripe // L):
        tbuf[pl.ds(k * L, L)] = zeros_f
        abuf[pl.ds(k * L, L)] = zeros_i
        mbuf[pl.ds(k * L, L)] = zeros_i

    cp_bi.wait()
    cp_pa.wait()
    cp_fa.wait()
    cp_pt.wait()

    for k in range(pad // L):
        off = step + k * L
        bi_v[pl.ds(off, L)] = zeros_i - 1
        pa_v[pl.ds(off, L)] = zeros_i
        fa_v[pl.ds(off, L)] = zeros_i
        pt_v[pl.ds(off, L)] = zeros_f

    def bsearch(target):
        def body(_, st):
            lo, hi = st
            mid = (lo + hi) // 2
            v = plsc.load_gather(bi_v, [zeros_i + mid])[0]
            go_right = v < target
            return (jnp.where(go_right, mid + 1, lo),
                    jnp.where(go_right, hi, mid))
        lo, _ = lax.fori_loop(0, 14, body, (jnp.int32(0), jnp.int32(step)))
        return lo

    lo = bsearch(wlo)
    hi = bsearch(whi)

    vs = lo // L
    nv = (hi + (L - 1)) // L - vs
    nb = (nv + (BLK_V - 1)) // BLK_V

    def block(b, _):
        base = (vs + b * BLK_V) * L
        for j in range(BLK_V):
            off = base + j * L
            pa16 = pa_v[pl.ds(off, L)]
            fa16 = fa_v[pl.ds(off, L)]
            bi16 = bi_v[pl.ds(off, L)]
            idx_dist[pl.ds(j * L, L)] = pa16 * seq + fa16
            rbase = (jnp.maximum(bi16, 0) * seq + fa16) * feat
            idx_rise[pl.ds(j * L, L)] = rbase + RISE_IDX
            idx_dur[pl.ds(j * L, L)] = rbase + VIS_IDX
        cp_d = pltpu.async_copy(dist_hbm.at[idx_dist], dist_vals, sem0)
        cp_r = pltpu.async_copy(raw_hbm.at[idx_rise], rise_vals, sem1)
        cp_v = pltpu.async_copy(raw_hbm.at[idx_dur], dur_vals, sem2)
        cp_d.wait()
        cp_r.wait()
        cp_v.wait()
        for j in range(BLK_V):
            off = base + j * L
            arrive = dist_vals[pl.ds(j * L, L)] + pt_v[pl.ds(off, L)]
            rise = rise_vals[pl.ds(j * L, L)]
            dur = dur_vals[pl.ds(j * L, L)]
            new_t = jnp.maximum(arrive, rise) + dur
            bi16 = bi_v[pl.ds(off, L)]
            bnext = plsc.load_gather(bi_v, [iota + (off + 1)])
            valid = ((bi16 != bnext) & (bi16 >= wlo) & (bi16 < whi))
            lidx = jnp.clip(bi16 - wlo, 0, stripe - 1)
            fa16 = fa_v[pl.ds(off, L)]
            plsc.store_scatter(tbuf, [lidx], new_t, mask=valid)
            plsc.store_scatter(abuf, [lidx], fa16, mask=valid)
            plsc.store_scatter(mbuf, [lidx], ones_i, mask=valid)
        return 0

    lax.fori_loop(0, nb, block, 0)

    pltpu.sync_copy(tbuf, time_out.at[pl.ds(wlo, stripe)])
    pltpu.sync_copy(abuf, act_out.at[pl.ds(wlo, stripe)])
    pltpu.sync_copy(mbuf, mask_out.at[pl.ds(wlo, stripe)])


def kernel(raw_inputs_b, dist_mat, present_time, pres_action, future_action,
           batch_idx, batch_size):
    batch, seq, feat = raw_inputs_b.shape
    step = pres_action.shape[0]
    pad = 2 * BLK

    raw_flat = raw_inputs_b.reshape(-1)
    dist_flat = dist_mat.reshape(-1)
    pt_flat = present_time.reshape(-1)

    body = functools.partial(_sc_body, step, batch, seq, feat)
    time_o, act_o, mask_o = pl.kernel(
        body,
        out_type=[
            jax.ShapeDtypeStruct((batch,), jnp.float32),
            jax.ShapeDtypeStruct((batch,), jnp.int32),
            jax.ShapeDtypeStruct((batch,), jnp.int32),
        ],
        mesh=plsc.VectorSubcoreMesh(core_axis_name="c", subcore_axis_name="s",
                                    num_cores=NC, num_subcores=NS),
        compiler_params=pltpu.CompilerParams(needs_layout_passes=False),
        scratch_types=[
            pltpu.VMEM((step + pad,), jnp.int32),
            pltpu.VMEM((step + pad,), jnp.int32),
            pltpu.VMEM((step + pad,), jnp.int32),
            pltpu.VMEM((step + pad,), jnp.float32),
            pltpu.VMEM((BLK,), jnp.int32),
            pltpu.VMEM((BLK,), jnp.int32),
            pltpu.VMEM((BLK,), jnp.int32),
            pltpu.VMEM((BLK,), jnp.float32),
            pltpu.VMEM((BLK,), jnp.float32),
            pltpu.VMEM((BLK,), jnp.float32),
            pltpu.VMEM((batch // NW,), jnp.float32),
            pltpu.VMEM((batch // NW,), jnp.int32),
            pltpu.VMEM((batch // NW,), jnp.int32),
            pltpu.SemaphoreType.DMA,
            pltpu.SemaphoreType.DMA,
            pltpu.SemaphoreType.DMA,
        ],
    )(raw_flat, dist_flat, pt_flat, pres_action, future_action, batch_idx)

    return (time_o.reshape(batch, 1),
            act_o,
            mask_o.astype(jnp.bool_).reshape(batch, 1))


# speedup vs baseline: 32.3066x; 32.3066x over previous
import functools

import jax
import jax.numpy as jnp
from jax import lax
from jax.experimental import pallas as pl
from jax.experimental.pallas import tpu as pltpu
from jax.experimental.pallas import tpu_sc as plsc

RISE_IDX = 0
VIS_IDX = 2

NC = 2
NS = 16
L = 16
NW = NC * NS

BLK_V = 8
BLK = BLK_V * L


def _sc_body(step, batch, seq, feat,
             rise_hbm, dur_hbm, dist_hbm, pt_hbm, pa_hbm, fa_hbm, bi_hbm,
             time_out, act_out, mask_out,
             bi_v, pa_v, fa_v, pt_v,
             idx_dist, idx_raw, dist_vals, rise_vals, dur_vals,
             tbuf, abuf, mbuf, sem0, sem1, sem2):
    stripe = batch // NW
    wid = lax.axis_index("s") * NC + lax.axis_index("c")
    wlo = wid * stripe
    whi = wlo + stripe
    pad = bi_v.shape[0] - step

    cp_bi = pltpu.async_copy(bi_hbm, bi_v.at[pl.ds(0, step)], sem0)
    cp_pa = pltpu.async_copy(pa_hbm, pa_v.at[pl.ds(0, step)], sem0)
    cp_fa = pltpu.async_copy(fa_hbm, fa_v.at[pl.ds(0, step)], sem0)
    cp_pt = pltpu.async_copy(pt_hbm, pt_v.at[pl.ds(0, step)], sem0)

    zeros_i = jnp.zeros((L,), jnp.int32)
    zeros_f = jnp.zeros((L,), jnp.float32)
    ones_i = jnp.ones((L,), jnp.int32)
    iota = lax.iota(jnp.int32, L)

    for k in range(stripe // L):
        tbuf[pl.ds(k * L, L)] = zeros_f
        abuf[pl.ds(k * L, L)] = zeros_i
        mbuf[pl.ds(k * L, L)] = zeros_i

    cp_bi.wait()
    cp_pa.wait()
    cp_fa.wait()
    cp_pt.wait()

    for k in range(pad // L):
        off = step + k * L
        bi_v[pl.ds(off, L)] = zeros_i - 1
        pa_v[pl.ds(off, L)] = zeros_i
        fa_v[pl.ds(off, L)] = zeros_i
        pt_v[pl.ds(off, L)] = zeros_f

    def bsearch(target):
        def body(_, st):
            lo, hi = st
            mid = (lo + hi) // 2
            v = plsc.load_gather(bi_v, [zeros_i + mid])[0]
            go_right = v < target
            return (jnp.where(go_right, mid + 1, lo),
                    jnp.where(go_right, hi, mid))
        lo, _ = lax.fori_loop(0, 14, body, (jnp.int32(0), jnp.int32(step)))
        return lo

    lo = bsearch(wlo)
    hi = bsearch(whi)

    vs = lo // L
    nv = (hi + (L - 1)) // L - vs
    nb = (nv + (BLK_V - 1)) // BLK_V

    def block(b, _):
        base = (vs + b * BLK_V) * L
        for j in range(BLK_V):
            off = base + j * L
            pa16 = pa_v[pl.ds(off, L)]
            fa16 = fa_v[pl.ds(off, L)]
            bi16 = bi_v[pl.ds(off, L)]
            idx_dist[pl.ds(j * L, L)] = pa16 * seq + fa16
            idx_raw[pl.ds(j * L, L)] = jnp.maximum(bi16, 0) * seq + fa16
        cp_d = pltpu.async_copy(dist_hbm.at[idx_dist], dist_vals, sem0)
        cp_r = pltpu.async_copy(rise_hbm.at[idx_raw], rise_vals, sem1)
        cp_v = pltpu.async_copy(dur_hbm.at[idx_raw], dur_vals, sem2)
        cp_d.wait()
        cp_r.wait()
        cp_v.wait()
        for j in range(BLK_V):
            off = base + j * L
            arrive = dist_vals[pl.ds(j * L, L)] + pt_v[pl.ds(off, L)]
            rise = rise_vals[pl.ds(j * L, L)]
            dur = dur_vals[pl.ds(j * L, L)]
            new_t = jnp.maximum(arrive, rise) + dur
            bi16 = bi_v[pl.ds(off, L)]
            bnext = plsc.load_gather(bi_v, [iota + (off + 1)])
            valid = ((bi16 != bnext) & (bi16 >= wlo) & (bi16 < whi))
            lidx = jnp.clip(bi16 - wlo, 0, stripe - 1)
            fa16 = fa_v[pl.ds(off, L)]
            plsc.store_scatter(tbuf, [lidx], new_t, mask=valid)
            plsc.store_scatter(abuf, [lidx], fa16, mask=valid)
            plsc.store_scatter(mbuf, [lidx], ones_i, mask=valid)
        return 0

    lax.fori_loop(0, nb, block, 0)

    pltpu.sync_copy(tbuf, time_out.at[pl.ds(wlo, stripe)])
    pltpu.sync_copy(abuf, act_out.at[pl.ds(wlo, stripe)])
    pltpu.sync_copy(mbuf, mask_out.at[pl.ds(wlo, stripe)])


def kernel(raw_inputs_b, dist_mat, present_time, pres_action, future_action,
           batch_idx, batch_size):
    batch, seq, feat = raw_inputs_b.shape
    step = pres_action.shape[0]
    pad = 2 * BLK

    rise_flat = raw_inputs_b[:, :, RISE_IDX].reshape(-1)
    dur_flat = raw_inputs_b[:, :, VIS_IDX].reshape(-1)
    dist_flat = dist_mat.reshape(-1)
    pt_flat = present_time.reshape(-1)

    body = functools.partial(_sc_body, step, batch, seq, feat)
    time_o, act_o, mask_o = pl.kernel(
        body,
        out_type=[
            jax.ShapeDtypeStruct((batch,), jnp.float32),
            jax.ShapeDtypeStruct((batch,), jnp.int32),
            jax.ShapeDtypeStruct((batch,), jnp.int32),
        ],
        mesh=plsc.VectorSubcoreMesh(core_axis_name="c", subcore_axis_name="s",
                                    num_cores=NC, num_subcores=NS),
        compiler_params=pltpu.CompilerParams(needs_layout_passes=False),
        scratch_types=[
            pltpu.VMEM((step + pad,), jnp.int32),
            pltpu.VMEM((step + pad,), jnp.int32),
            pltpu.VMEM((step + pad,), jnp.int32),
            pltpu.VMEM((step + pad,), jnp.float32),
            pltpu.VMEM((BLK,), jnp.int32),
            pltpu.VMEM((BLK,), jnp.int32),
            pltpu.VMEM((BLK,), jnp.float32),
            pltpu.VMEM((BLK,), jnp.float32),
            pltpu.VMEM((BLK,), jnp.float32),
            pltpu.VMEM((batch // NW,), jnp.float32),
            pltpu.VMEM((batch // NW,), jnp.int32),
            pltpu.VMEM((batch // NW,), jnp.int32),
            pltpu.SemaphoreType.DMA,
            pltpu.SemaphoreType.DMA,
            pltpu.SemaphoreType.DMA,
        ],
    )(rise_flat, dur_flat, dist_flat, pt_flat, pres_action, future_action,
      batch_idx)

    return (time_o.reshape(batch, 1),
            act_o,
            mask_o.astype(jnp.bool_).reshape(batch, 1))


# speedup vs baseline: 116.8933x; 3.6183x over previous
import functools

import jax
import jax.numpy as jnp
from jax import lax
from jax.experimental import pallas as pl
from jax.experimental.pallas import tpu as pltpu
from jax.experimental.pallas import tpu_sc as plsc

RISE_IDX = 0
VIS_IDX = 2

NC = 2
NS = 16
L = 16
NW = NC * NS

BLK_V = 8
BLK = BLK_V * L


def _sc_body(step, batch, seq, feat,
             raw_hbm, dist_hbm, pt_hbm, pa_hbm, fa_hbm, bi_hbm,
             time_out, act_out, mask_out,
             bi_v, pa_v, fa_v, pt_v,
             idx_dist, idx_rise, idx_dur, dist_vals, rise_vals, dur_vals,
             tbuf, abuf, mbuf, sem0, sem1, sem2):
    stripe = batch // NW
    wid = lax.axis_index("s") * NC + lax.axis_index("c")
    wlo = wid * stripe
    whi = wlo + stripe
    pad = bi_v.shape[0] - step

    cp_bi = pltpu.async_copy(bi_hbm, bi_v.at[pl.ds(0, step)], sem0)
    cp_pa = pltpu.async_copy(pa_hbm, pa_v.at[pl.ds(0, step)], sem0)
    cp_fa = pltpu.async_copy(fa_hbm, fa_v.at[pl.ds(0, step)], sem0)
    cp_pt = pltpu.async_copy(pt_hbm, pt_v.at[pl.ds(0, step)], sem0)

    zeros_i = jnp.zeros((L,), jnp.int32)
    zeros_f = jnp.zeros((L,), jnp.float32)
    ones_i = jnp.ones((L,), jnp.int32)
    iota = lax.iota(jnp.int32, L)

    for k in range(stripe // L):
        tbuf[pl.ds(k * L, L)] = zeros_f
        abuf[pl.ds(k * L, L)] = zeros_i
        mbuf[pl.ds(k * L, L)] = zeros_i

    cp_bi.wait()
    cp_pa.wait()
    cp_fa.wait()
    cp_pt.wait()

    for k in range(pad // L):
        off = step + k * L
        bi_v[pl.ds(off, L)] = zeros_i - 1
        pa_v[pl.ds(off, L)] = zeros_i
        fa_v[pl.ds(off, L)] = zeros_i
        pt_v[pl.ds(off, L)] = zeros_f

    def bsearch(target):
        def body(_, st):
            lo, hi = st
            mid = (lo + hi) // 2
            v = plsc.load_gather(bi_v, [zeros_i + mid])[0]
            go_right = v < target
            return (jnp.where(go_right, mid + 1, lo),
                    jnp.where(go_right, hi, mid))
        lo, _ = lax.fori_loop(0, 14, body, (jnp.int32(0), jnp.int32(step)))
        return lo

    lo = bsearch(wlo)
    hi = bsearch(whi)

    vs = lo // L
    nv = (hi + (L - 1)) // L - vs
    nb = (nv + (BLK_V - 1)) // BLK_V

    def block(b, _):
        base = (vs + b * BLK_V) * L
        for j in range(BLK_V):
            off = base + j * L
            pa16 = pa_v[pl.ds(off, L)]
            fa16 = fa_v[pl.ds(off, L)]
            bi16 = bi_v[pl.ds(off, L)]
            idx_dist[pl.ds(j * L, L)] = (
                (pa16 >> 3) * 4096 + (fa16 >> 7) * 1024
                + (pa16 & 7) * 128 + (fa16 & 127))
            bc = jnp.maximum(bi16, 0)
            tidx = ((bc >> 3) * 4096 + (fa16 >> 7) * 1024
                    + (bc & 7) * 128 + (fa16 & 127))
            idx_rise[pl.ds(j * L, L)] = tidx + RISE_IDX * batch * seq
            idx_dur[pl.ds(j * L, L)] = tidx + VIS_IDX * batch * seq
        cp_d = pltpu.async_copy(dist_hbm.at[idx_dist], dist_vals, sem0)
        cp_r = pltpu.async_copy(raw_hbm.at[idx_rise], rise_vals, sem1)
        cp_v = pltpu.async_copy(raw_hbm.at[idx_dur], dur_vals, sem2)
        cp_d.wait()
        cp_r.wait()
        cp_v.wait()
        for j in range(BLK_V):
            off = base + j * L
            arrive = dist_vals[pl.ds(j * L, L)] + pt_v[pl.ds(off, L)]
            rise = rise_vals[pl.ds(j * L, L)]
            dur = dur_vals[pl.ds(j * L, L)]
            new_t = jnp.maximum(arrive, rise) + dur
            bi16 = bi_v[pl.ds(off, L)]
            bnext = plsc.load_gather(bi_v, [iota + (off + 1)])
            valid = ((bi16 != bnext) & (bi16 >= wlo) & (bi16 < whi))
            lidx = jnp.clip(bi16 - wlo, 0, stripe - 1)
            fa16 = fa_v[pl.ds(off, L)]
            plsc.store_scatter(tbuf, [lidx], new_t, mask=valid)
            plsc.store_scatter(abuf, [lidx], fa16, mask=valid)
            plsc.store_scatter(mbuf, [lidx], ones_i, mask=valid)
        return 0

    lax.fori_loop(0, nb, block, 0)

    pltpu.sync_copy(tbuf, time_out.at[pl.ds(wlo, stripe)])
    pltpu.sync_copy(abuf, act_out.at[pl.ds(wlo, stripe)])
    pltpu.sync_copy(mbuf, mask_out.at[pl.ds(wlo, stripe)])


def kernel(raw_inputs_b, dist_mat, present_time, pres_action, future_action,
           batch_idx, batch_size):
    batch, seq, feat = raw_inputs_b.shape
    step = pres_action.shape[0]
    pad = 2 * BLK

    raw_tiles = (raw_inputs_b
                 .transpose(2, 0, 1)
                 .reshape(feat, batch // 8, 8, seq // 128, 128)
                 .transpose(0, 1, 3, 2, 4)
                 .reshape(-1))
    dist_tiles = (dist_mat
                  .reshape(seq // 8, 8, seq // 128, 128)
                  .transpose(0, 2, 1, 3)
                  .reshape(-1))
    pt_flat = present_time.reshape(-1)

    body = functools.partial(_sc_body, step, batch, seq, feat)
    time_o, act_o, mask_o = pl.kernel(
        body,
        out_type=[
            jax.ShapeDtypeStruct((batch,), jnp.float32),
            jax.ShapeDtypeStruct((batch,), jnp.int32),
            jax.ShapeDtypeStruct((batch,), jnp.int32),
        ],
        mesh=plsc.VectorSubcoreMesh(core_axis_name="c", subcore_axis_name="s",
                                    num_cores=NC, num_subcores=NS),
        compiler_params=pltpu.CompilerParams(needs_layout_passes=False),
        scratch_types=[
            pltpu.VMEM((step + pad,), jnp.int32),
            pltpu.VMEM((step + pad,), jnp.int32),
            pltpu.VMEM((step + pad,), jnp.int32),
            pltpu.VMEM((step + pad,), jnp.float32),
            pltpu.VMEM((BLK,), jnp.int32),
            pltpu.VMEM((BLK,), jnp.int32),
            pltpu.VMEM((BLK,), jnp.int32),
            pltpu.VMEM((BLK,), jnp.float32),
            pltpu.VMEM((BLK,), jnp.float32),
            pltpu.VMEM((BLK,), jnp.float32),
            pltpu.VMEM((batch // NW,), jnp.float32),
            pltpu.VMEM((batch // NW,), jnp.int32),
            pltpu.VMEM((batch // NW,), jnp.int32),
            pltpu.SemaphoreType.DMA,
            pltpu.SemaphoreType.DMA,
            pltpu.SemaphoreType.DMA,
        ],
    )(raw_tiles, dist_tiles, pt_flat, pres_action, future_action, batch_idx)

    return (time_o.reshape(batch, 1),
            act_o,
            mask_o.astype(jnp.bool_).reshape(batch, 1))


# speedup vs baseline: 119.0902x; 1.0188x over previous
import functools

import jax
import jax.numpy as jnp
from jax import lax
from jax.experimental import pallas as pl
from jax.experimental.pallas import tpu as pltpu
from jax.experimental.pallas import tpu_sc as plsc

RISE_IDX = 0
VIS_IDX = 2

NC = 2
NS = 16
L = 16
NW = NC * NS

BLK_V = 8
BLK = BLK_V * L


def _sc_body(step, batch, seq, feat,
             raw_hbm, dist_hbm, pt_hbm, pa_hbm, fa_hbm, bi_hbm,
             time_out, act_out, mask_out,
             bi_v, pa_v, fa_v, pt_v,
             idx_dist, idx_rise, idx_dur, dist_vals, rise_vals, dur_vals,
             tbuf, abuf, mbuf, sem0, sem1, sem2):
    stripe = batch // NW
    wid = lax.axis_index("s") * NC + lax.axis_index("c")
    wlo = wid * stripe
    whi = wlo + stripe
    pad = bi_v.shape[0] - step

    cp_bi = pltpu.async_copy(bi_hbm, bi_v.at[pl.ds(0, step)], sem0)
    cp_pa = pltpu.async_copy(pa_hbm, pa_v.at[pl.ds(0, step)], sem0)
    cp_fa = pltpu.async_copy(fa_hbm, fa_v.at[pl.ds(0, step)], sem0)
    cp_pt = pltpu.async_copy(pt_hbm, pt_v.at[pl.ds(0, step)], sem0)

    zeros_i = jnp.zeros((L,), jnp.int32)
    zeros_f = jnp.zeros((L,), jnp.float32)
    ones_i = jnp.ones((L,), jnp.int32)
    iota = lax.iota(jnp.int32, L)

    def zero_body(k, _):
        tbuf[pl.ds(k * L, L)] = zeros_f
        abuf[pl.ds(k * L, L)] = zeros_i
        mbuf[pl.ds(k * L, L)] = zeros_i
        return 0
    lax.fori_loop(0, stripe // L, zero_body, 0)

    cp_bi.wait()
    cp_pa.wait()
    cp_fa.wait()
    cp_pt.wait()

    def pad_body(k, _):
        off = step + k * L
        bi_v[pl.ds(off, L)] = zeros_i - 1
        pa_v[pl.ds(off, L)] = zeros_i
        fa_v[pl.ds(off, L)] = zeros_i
        pt_v[pl.ds(off, L)] = zeros_f
        return 0
    lax.fori_loop(0, pad // L, pad_body, 0)

    def bsearch(target):
        def body(_, st):
            lo, hi = st
            mid = (lo + hi) // 2
            v = plsc.load_gather(bi_v, [zeros_i + mid])[0]
            go_right = v < target
            return (jnp.where(go_right, mid + 1, lo),
                    jnp.where(go_right, hi, mid))
        lo, _ = lax.fori_loop(0, 14, body, (jnp.int32(0), jnp.int32(step)))
        return lo

    lo = bsearch(wlo)
    hi = bsearch(whi)

    vs = lo // L
    nv = (hi + (L - 1)) // L - vs
    nb = (nv + (BLK_V - 1)) // BLK_V

    def block(b, _):
        base = (vs + b * BLK_V) * L

        def build(j, _):
            off = base + j * L
            pa16 = pa_v[pl.ds(off, L)]
            fa16 = fa_v[pl.ds(off, L)]
            bi16 = bi_v[pl.ds(off, L)]
            idx_dist[pl.ds(j * L, L)] = (
                (pa16 >> 3) * 4096 + (fa16 >> 7) * 1024
                + (pa16 & 7) * 128 + (fa16 & 127))
            bc = jnp.maximum(bi16, 0)
            tidx = ((bc >> 3) * 4096 + (fa16 >> 7) * 1024
                    + (bc & 7) * 128 + (fa16 & 127))
            idx_rise[pl.ds(j * L, L)] = tidx + RISE_IDX * batch * seq
            idx_dur[pl.ds(j * L, L)] = tidx + VIS_IDX * batch * seq
            return 0
        lax.fori_loop(0, BLK_V, build, 0)

        cp_d = pltpu.async_copy(dist_hbm.at[idx_dist], dist_vals, sem0)
        cp_r = pltpu.async_copy(raw_hbm.at[idx_rise], rise_vals, sem1)
        cp_v = pltpu.async_copy(raw_hbm.at[idx_dur], dur_vals, sem2)
        cp_d.wait()
        cp_r.wait()
        cp_v.wait()

        def compute(j, _):
            off = base + j * L
            arrive = dist_vals[pl.ds(j * L, L)] + pt_v[pl.ds(off, L)]
            rise = rise_vals[pl.ds(j * L, L)]
            dur = dur_vals[pl.ds(j * L, L)]
            new_t = jnp.maximum(arrive, rise) + dur
            bi16 = bi_v[pl.ds(off, L)]
            bnext = plsc.load_gather(bi_v, [iota + (off + 1)])
            valid = ((bi16 != bnext) & (bi16 >= wlo) & (bi16 < whi))
            lidx = jnp.clip(bi16 - wlo, 0, stripe - 1)
            fa16 = fa_v[pl.ds(off, L)]
            plsc.store_scatter(tbuf, [lidx], new_t, mask=valid)
            plsc.store_scatter(abuf, [lidx], fa16, mask=valid)
            plsc.store_scatter(mbuf, [lidx], ones_i, mask=valid)
            return 0
        lax.fori_loop(0, BLK_V, compute, 0)
        return 0

    lax.fori_loop(0, nb, block, 0)

    pltpu.sync_copy(tbuf, time_out.at[pl.ds(wlo, stripe)])
    pltpu.sync_copy(abuf, act_out.at[pl.ds(wlo, stripe)])
    pltpu.sync_copy(mbuf, mask_out.at[pl.ds(wlo, stripe)])


def kernel(raw_inputs_b, dist_mat, present_time, pres_action, future_action,
           batch_idx, batch_size):
    batch, seq, feat = raw_inputs_b.shape
    step = pres_action.shape[0]
    pad = 2 * BLK

    raw_tiles = (raw_inputs_b
                 .transpose(2, 0, 1)
                 .reshape(feat, batch // 8, 8, seq // 128, 128)
                 .transpose(0, 1, 3, 2, 4)
                 .reshape(-1))
    dist_tiles = (dist_mat
                  .reshape(seq // 8, 8, seq // 128, 128)
                  .transpose(0, 2, 1, 3)
                  .reshape(-1))
    pt_flat = present_time.reshape(-1)

    body = functools.partial(_sc_body, step, batch, seq, feat)
    time_o, act_o, mask_o = pl.kernel(
        body,
        out_type=[
            jax.ShapeDtypeStruct((batch,), jnp.float32),
            jax.ShapeDtypeStruct((batch,), jnp.int32),
            jax.ShapeDtypeStruct((batch,), jnp.int32),
        ],
        mesh=plsc.VectorSubcoreMesh(core_axis_name="c", subcore_axis_name="s",
                                    num_cores=NC, num_subcores=NS),
        compiler_params=pltpu.CompilerParams(needs_layout_passes=False),
        scratch_types=[
            pltpu.VMEM((step + pad,), jnp.int32),
            pltpu.VMEM((step + pad,), jnp.int32),
            pltpu.VMEM((step + pad,), jnp.int32),
            pltpu.VMEM((step + pad,), jnp.float32),
            pltpu.VMEM((BLK,), jnp.int32),
            pltpu.VMEM((BLK,), jnp.int32),
            pltpu.VMEM((BLK,), jnp.int32),
            pltpu.VMEM((BLK,), jnp.float32),
            pltpu.VMEM((BLK,), jnp.float32),
            pltpu.VMEM((BLK,), jnp.float32),
            pltpu.VMEM((batch // NW,), jnp.float32),
            pltpu.VMEM((batch // NW,), jnp.int32),
            pltpu.VMEM((batch // NW,), jnp.int32),
            pltpu.SemaphoreType.DMA,
            pltpu.SemaphoreType.DMA,
            pltpu.SemaphoreType.DMA,
        ],
    )(raw_tiles, dist_tiles, pt_flat, pres_action, future_action, batch_idx)

    return (time_o.reshape(batch, 1),
            act_o,
            mask_o.astype(jnp.bool_).reshape(batch, 1))


# speedup vs baseline: 121.5961x; 1.0210x over previous
import functools

import jax
import jax.numpy as jnp
from jax import lax
from jax.experimental import pallas as pl
from jax.experimental.pallas import tpu as pltpu
from jax.experimental.pallas import tpu_sc as plsc

RISE_IDX = 0
VIS_IDX = 2

NC = 2
NS = 16
L = 16
NW = NC * NS

BLK = 128
SUP_K = 3
SUP_V = SUP_K * BLK // L
SUP = SUP_K * BLK


def _sc_body(step, batch, seq, feat,
             raw_hbm, dist_hbm, pt_hbm, pa_hbm, fa_hbm, bi_hbm,
             time_out, act_out, mask_out,
             bi_v, pa_v, fa_v, pt_v,
             idx_dist, idx_rise, idx_dur, dist_vals, rise_vals, dur_vals,
             tbuf, abuf, mbuf, sem0, sem1, sem2):
    stripe = batch // NW
    wid = lax.axis_index("s") * NC + lax.axis_index("c")
    wlo = wid * stripe
    whi = wlo + stripe
    pad = bi_v.shape[0] - step

    cp_bi = pltpu.async_copy(bi_hbm, bi_v.at[pl.ds(0, step)], sem0)
    cp_pa = pltpu.async_copy(pa_hbm, pa_v.at[pl.ds(0, step)], sem0)
    cp_fa = pltpu.async_copy(fa_hbm, fa_v.at[pl.ds(0, step)], sem0)
    cp_pt = pltpu.async_copy(pt_hbm, pt_v.at[pl.ds(0, step)], sem0)

    zeros_i = jnp.zeros((L,), jnp.int32)
    zeros_f = jnp.zeros((L,), jnp.float32)
    ones_i = jnp.ones((L,), jnp.int32)
    iota = lax.iota(jnp.int32, L)

    def zero_body(k, _):
        tbuf[pl.ds(k * L, L)] = zeros_f
        abuf[pl.ds(k * L, L)] = zeros_i
        mbuf[pl.ds(k * L, L)] = zeros_i
        return 0
    lax.fori_loop(0, stripe // L, zero_body, 0)

    cp_bi.wait()

    def pad_body(k, _):
        off = step + k * L
        bi_v[pl.ds(off, L)] = zeros_i - 1
        return 0
    lax.fori_loop(0, pad // L, pad_body, 0)

    def bsearch(target):
        def body(_, st):
            lo, hi = st
            mid = (lo + hi) // 2
            v = plsc.load_gather(bi_v, [zeros_i + mid])[0]
            go_right = v < target
            return (jnp.where(go_right, mid + 1, lo),
                    jnp.where(go_right, hi, mid))
        lo, _ = lax.fori_loop(0, 14, body, (jnp.int32(0), jnp.int32(step)))
        return lo

    lo = bsearch(wlo)
    hi = bsearch(whi)

    cp_pa.wait()
    cp_fa.wait()
    cp_pt.wait()

    def pad_body2(k, _):
        off = step + k * L
        pa_v[pl.ds(off, L)] = zeros_i
        fa_v[pl.ds(off, L)] = zeros_i
        pt_v[pl.ds(off, L)] = zeros_f
        return 0
    lax.fori_loop(0, pad // L, pad_body2, 0)

    vs = lo // L
    nv = (hi + (L - 1)) // L - vs
    nb = (nv + (SUP_V - 1)) // SUP_V

    def block(b, _):
        base = (vs + b * SUP_V) * L

        def build(j, _):
            off = base + j * L
            pa16 = pa_v[pl.ds(off, L)]
            fa16 = fa_v[pl.ds(off, L)]
            bi16 = bi_v[pl.ds(off, L)]
            idx_dist[pl.ds(j * L, L)] = (
                (pa16 >> 3) * 4096 + (fa16 >> 7) * 1024
                + (pa16 & 7) * 128 + (fa16 & 127))
            bc = jnp.maximum(bi16, 0)
            tidx = ((bc >> 3) * 4096 + (fa16 >> 7) * 1024
                    + (bc & 7) * 128 + (fa16 & 127))
            idx_rise[pl.ds(j * L, L)] = tidx + RISE_IDX * batch * seq
            idx_dur[pl.ds(j * L, L)] = tidx + VIS_IDX * batch * seq
            return 0
        lax.fori_loop(0, SUP_V, build, 0)

        cps = []
        for k in range(SUP_K):
            s = pl.ds(k * BLK, BLK)
            cps.append(pltpu.async_copy(
                dist_hbm.at[idx_dist.at[s]], dist_vals.at[s], sem0))
            cps.append(pltpu.async_copy(
                raw_hbm.at[idx_rise.at[s]], rise_vals.at[s], sem1))
            cps.append(pltpu.async_copy(
                raw_hbm.at[idx_dur.at[s]], dur_vals.at[s], sem2))
        for cp in cps:
            cp.wait()

        def compute(j, _):
            off = base + j * L
            arrive = dist_vals[pl.ds(j * L, L)] + pt_v[pl.ds(off, L)]
            rise = rise_vals[pl.ds(j * L, L)]
            dur = dur_vals[pl.ds(j * L, L)]
            new_t = jnp.maximum(arrive, rise) + dur
            bi16 = bi_v[pl.ds(off, L)]
            bnext = plsc.load_gather(bi_v, [iota + (off + 1)])
            valid = ((bi16 != bnext) & (bi16 >= wlo) & (bi16 < whi))
            lidx = jnp.clip(bi16 - wlo, 0, stripe - 1)
            fa16 = fa_v[pl.ds(off, L)]
            plsc.store_scatter(tbuf, [lidx], new_t, mask=valid)
            plsc.store_scatter(abuf, [lidx], fa16, mask=valid)
            plsc.store_scatter(mbuf, [lidx], ones_i, mask=valid)
            return 0
        lax.fori_loop(0, SUP_V, compute, 0)
        return 0

    lax.fori_loop(0, nb, block, 0)

    pltpu.sync_copy(tbuf, time_out.at[pl.ds(wlo, stripe)])
    pltpu.sync_copy(abuf, act_out.at[pl.ds(wlo, stripe)])
    pltpu.sync_copy(mbuf, mask_out.at[pl.ds(wlo, stripe)])


def kernel(raw_inputs_b, dist_mat, present_time, pres_action, future_action,
           batch_idx, batch_size):
    batch, seq, feat = raw_inputs_b.shape
    step = pres_action.shape[0]
    pad = SUP + 2 * L

    raw_tiles = (raw_inputs_b
                 .transpose(2, 0, 1)
                 .reshape(feat, batch // 8, 8, seq // 128, 128)
                 .transpose(0, 1, 3, 2, 4)
                 .reshape(-1))
    dist_tiles = (dist_mat
                  .reshape(seq // 8, 8, seq // 128, 128)
                  .transpose(0, 2, 1, 3)
                  .reshape(-1))
    pt_flat = present_time.reshape(-1)

    body = functools.partial(_sc_body, step, batch, seq, feat)
    time_o, act_o, mask_o = pl.kernel(
        body,
        out_type=[
            jax.ShapeDtypeStruct((batch,), jnp.float32),
            jax.ShapeDtypeStruct((batch,), jnp.int32),
            jax.ShapeDtypeStruct((batch,), jnp.int32),
        ],
        mesh=plsc.VectorSubcoreMesh(core_axis_name="c", subcore_axis_name="s",
                                    num_cores=NC, num_subcores=NS),
        compiler_params=pltpu.CompilerParams(needs_layout_passes=False),
        scratch_types=[
            pltpu.VMEM((step + pad,), jnp.int32),
            pltpu.VMEM((step + pad,), jnp.int32),
            pltpu.VMEM((step + pad,), jnp.int32),
            pltpu.VMEM((step + pad,), jnp.float32),
            pltpu.VMEM((SUP,), jnp.int32),
            pltpu.VMEM((SUP,), jnp.int32),
            pltpu.VMEM((SUP,), jnp.int32),
            pltpu.VMEM((SUP,), jnp.float32),
            pltpu.VMEM((SUP,), jnp.float32),
            pltpu.VMEM((SUP,), jnp.float32),
            pltpu.VMEM((batch // NW,), jnp.float32),
            pltpu.VMEM((batch // NW,), jnp.int32),
            pltpu.VMEM((batch // NW,), jnp.int32),
            pltpu.SemaphoreType.DMA,
            pltpu.SemaphoreType.DMA,
            pltpu.SemaphoreType.DMA,
        ],
    )(raw_tiles, dist_tiles, pt_flat, pres_action, future_action, batch_idx)

    return (time_o.reshape(batch, 1),
            act_o,
            mask_o.astype(jnp.bool_).reshape(batch, 1))
